# baseline (speedup 1.0000x reference)
import functools

import jax
import jax.numpy as jnp
from jax import lax
from jax.experimental import pallas as pl
from jax.experimental.pallas import tpu as pltpu
from jax.experimental.pallas import tpu_sc as plsc

BATCH = 4
CTX = 8192
EMB = 1024

NC, NS = 2, 16
NW = NC * NS
ROWS = BATCH * CTX
RPW = ROWS // NW
CHUNK = 16
CELTS = CHUNK * EMB
NCHUNK = RPW // CHUNK


def _sc_body(x_hbm, pos_hbm, o_hbm,
             xv0, xv1, pv0, pv1,
             sx0, sx1, sp0, sp1, so0, so1):
    wid = lax.axis_index("s") * NC + lax.axis_index("c")
    base = wid * (RPW * EMB)
    pbase = (wid % (NW // BATCH)) * (RPW * EMB)

    xv = (xv0, xv1)
    pv = (pv0, pv1)
    sx = (sx0, sx1)
    sp = (sp0, sp1)
    so = (so0, so1)

    def start_in(k, b):
        off = base + k * CELTS
        poff = pbase + k * CELTS
        pltpu.async_copy(x_hbm.at[pl.ds(off, CELTS)], xv[b], sx[b])
        pltpu.async_copy(pos_hbm.at[pl.ds(poff, CELTS)], pv[b], sp[b])

    start_in(0, 0)

    @pl.loop(0, NCHUNK, step=2)
    def _(k):
        for b in range(2):
            kk = k + b
            pltpu.make_async_copy(x_hbm.at[pl.ds(0, CELTS)], xv[b], sx[b]).wait()
            pltpu.make_async_copy(pos_hbm.at[pl.ds(0, CELTS)], pv[b], sp[b]).wait()

            @pl.when(kk >= 1)
            def _():
                pltpu.make_async_copy(
                    xv[1 - b], o_hbm.at[pl.ds(0, CELTS)], so[1 - b]).wait()

            @pl.when(kk + 1 < NCHUNK)
            def _():
                start_in(kk + 1, 1 - b)

            @pl.loop(0, CELTS // 16, unroll=8)
            def _(j):
                sl = pl.ds(j * 16, 16)
                xv[b][sl] = xv[b][sl] + pv[b][sl]

            pltpu.async_copy(xv[b], o_hbm.at[pl.ds(base + kk * CELTS, CELTS)],
                             so[b])

    pltpu.make_async_copy(
        xv[(NCHUNK - 1) % 2], o_hbm.at[pl.ds(0, CELTS)],
        so[(NCHUNK - 1) % 2]).wait()


@jax.jit
def _sc_add(x_flat, pos_flat):
    mesh = plsc.VectorSubcoreMesh(core_axis_name="c", subcore_axis_name="s")
    return pl.kernel(
        _sc_body,
        out_type=jax.ShapeDtypeStruct((ROWS * EMB,), jnp.float32),
        mesh=mesh,
        scratch_types=[
            pltpu.VMEM((CELTS,), jnp.float32),
            pltpu.VMEM((CELTS,), jnp.float32),
            pltpu.VMEM((CELTS,), jnp.float32),
            pltpu.VMEM((CELTS,), jnp.float32),
            pltpu.SemaphoreType.DMA,
            pltpu.SemaphoreType.DMA,
            pltpu.SemaphoreType.DMA,
            pltpu.SemaphoreType.DMA,
            pltpu.SemaphoreType.DMA,
            pltpu.SemaphoreType.DMA,
        ],
    )(x_flat, pos_flat)


def kernel(x, pos_table):
    x_flat = x.reshape(-1)
    pos_flat = pos_table.reshape(-1)
    out = _sc_add(x_flat, pos_flat)
    return out.reshape(x.shape)


# speedup vs baseline: 3.0261x; 3.0261x over previous
import jax
import jax.numpy as jnp
from jax.experimental import pallas as pl

RB = 1024


def _add_kernel(x_ref, pos_ref, o_ref):
    o_ref[...] = x_ref[...] + pos_ref[...]


def kernel(x, pos_table):
    batch, ctx, emb = x.shape
    rows = batch * ctx
    pblocks = ctx // RB
    x2 = x.reshape(rows, emb)
    out = pl.pallas_call(
        _add_kernel,
        grid=(rows // RB,),
        in_specs=[
            pl.BlockSpec((RB, emb), lambda i: (i, 0)),
            pl.BlockSpec((RB, emb), lambda i: (i % pblocks, 0)),
        ],
        out_specs=pl.BlockSpec((RB, emb), lambda i: (i, 0)),
        out_shape=jax.ShapeDtypeStruct((rows, emb), x.dtype),
    )(x2, pos_table)
    return out.reshape(x.shape)


# speedup vs baseline: 3.9753x; 1.3137x over previous
import jax
import jax.numpy as jnp
from jax.experimental import pallas as pl

BLK = 512


def _add_kernel(x_ref, pos_ref, o_ref):
    o_ref[...] = x_ref[...] + pos_ref[...][None, :, :]


def kernel(x, pos_table):
    batch, ctx, emb = x.shape
    grid = (ctx // BLK,)
    return pl.pallas_call(
        _add_kernel,
        grid=grid,
        in_specs=[
            pl.BlockSpec((batch, BLK, emb), lambda i: (0, i, 0)),
            pl.BlockSpec((BLK, emb), lambda i: (i, 0)),
        ],
        out_specs=pl.BlockSpec((batch, BLK, emb), lambda i: (0, i, 0)),
        out_shape=jax.ShapeDtypeStruct(x.shape, x.dtype),
    )(x, pos_table)
